# baseline (speedup 1.0000x reference)
import functools

import jax
import jax.numpy as jnp
from jax import lax
from jax.experimental import pallas as pl
from jax.experimental.pallas import tpu as pltpu
from jax.experimental.pallas import tpu_sc as plsc

N_FIELDS = 26
BATCH = 4096
L = 20
D = 64
VOCAB = 100000

FG = 2
NG = N_FIELDS // FG
SLAB_ROWS = FG * VOCAB

NC = 2
NS = 16
NW = NC * NS
BPW = BATCH // NW
CB = 32
CHUNKS_PER_FIELD = BPW // CB
TOT_CHUNKS = FG * CHUNKS_PER_FIELD
IPC = CB * L
IDX_ROWS = IPC // 128


def _sc_body(idx_hbm, table_hbm, offs_hbm, out_hbm,
             idx_v0, idx_v1, rows_v0, rows_v1, out_v, offs_v, sem0, sem1):
    c = lax.axis_index("c")
    s = lax.axis_index("s")
    wid = s * NC + c

    idx_vs = (idx_v0, idx_v1)
    rows_vs = (rows_v0, rows_v1)
    sems = (sem0, sem1)

    pltpu.sync_copy(offs_hbm, offs_v)

    def base_bag_of(t):
        f = t // CHUNKS_PER_FIELD
        cc = lax.rem(t, CHUNKS_PER_FIELD)
        return f, f * BATCH + wid * BPW + cc * CB

    def fire(t, ib):
        f, base_bag = base_bag_of(t)
        pltpu.sync_copy(idx_hbm.at[pl.ds(base_bag * L, IPC)], idx_vs[ib])
        off = offs_v[pl.ds(f * 16, 16)]
        for i in range(IPC // 16):
            sl = pl.ds(i * 16, 16)
            idx_vs[ib][sl] = idx_vs[ib][sl] + off
        for i in range(IDX_ROWS):
            pltpu.async_copy(table_hbm.at[idx_vs[ib].at[pl.ds(i * 128, 128)]],
                             rows_vs[ib].at[pl.ds(i * 128, 128)], sems[ib])

    def drain(ib):
        pltpu.make_async_copy(table_hbm.at[pl.ds(0, IPC)],
                              rows_vs[ib], sems[ib]).wait()

    def reduce_store(t, ib):
        _, base_bag = base_bag_of(t)
        rows_v = rows_vs[ib]

        @pl.loop(0, CB)
        def bag_loop(j):
            row0 = j * L
            for k in range(4):
                sl = pl.ds(k * 16, 16)
                a0 = rows_v[row0 + 0, sl] + rows_v[row0 + 1, sl]
                a1 = rows_v[row0 + 2, sl] + rows_v[row0 + 3, sl]
                a2 = rows_v[row0 + 4, sl] + rows_v[row0 + 5, sl]
                a3 = rows_v[row0 + 6, sl] + rows_v[row0 + 7, sl]
                for l in range(8, L, 4):
                    a0 = a0 + rows_v[row0 + l + 0, sl]
                    a1 = a1 + rows_v[row0 + l + 1, sl]
                    a2 = a2 + rows_v[row0 + l + 2, sl]
                    a3 = a3 + rows_v[row0 + l + 3, sl]
                out_v[j, sl] = (a0 + a1) + (a2 + a3)

        pltpu.sync_copy(out_v, out_hbm.at[pl.ds(base_bag, CB)])

    fire(0, 0)

    @pl.loop(0, TOT_CHUNKS, step=2)
    def chunk_loop(t):
        for b in range(2):
            tb = t + b
            nxt = tb + 1

            @pl.when(nxt < TOT_CHUNKS)
            def _():
                fire(nxt, 1 - b)

            drain(b)
            reduce_store(tb, b)


@jax.jit
def _fused_bag_sum(idx1d, table, offsets):
    mesh = plsc.VectorSubcoreMesh(core_axis_name="c", subcore_axis_name="s")
    fn = pl.kernel(
        _sc_body,
        out_type=jax.ShapeDtypeStruct((FG * BATCH, D), jnp.float32),
        mesh=mesh,
        compiler_params=pltpu.CompilerParams(use_tc_tiling_on_sc=False),
        scratch_types=[
            pltpu.VMEM((IPC,), jnp.int32),
            pltpu.VMEM((IPC,), jnp.int32),
            pltpu.VMEM((IPC, D), jnp.float32),
            pltpu.VMEM((IPC, D), jnp.float32),
            pltpu.VMEM((CB, D), jnp.float32),
            pltpu.VMEM((32,), jnp.int32),
            pltpu.SemaphoreType.DMA,
            pltpu.SemaphoreType.DMA,
        ],
    )
    outs = []
    for g in range(NG):
        idx_g = lax.slice(idx1d, (g * FG * BATCH * L,),
                          ((g + 1) * FG * BATCH * L,))
        tbl_g = lax.slice(table, (g * SLAB_ROWS, 0), ((g + 1) * SLAB_ROWS, D))
        offs_g = jnp.repeat(
            lax.slice(offsets, (g * FG,), ((g + 1) * FG,)) - g * SLAB_ROWS,
            16,
        )
        outs.append(fn(idx_g, tbl_g, offs_g))
    return jnp.concatenate(outs, axis=0)


def kernel(indices, table, offsets):
    idx1d = indices.reshape(-1)
    out = _fused_bag_sum(idx1d, table, offsets)
    return out.reshape(BATCH, N_FIELDS, D)


# speedup vs baseline: 1.0016x; 1.0016x over previous
import functools

import jax
import jax.numpy as jnp
from jax import lax
from jax.experimental import pallas as pl
from jax.experimental.pallas import tpu as pltpu
from jax.experimental.pallas import tpu_sc as plsc

N_FIELDS = 26
BATCH = 4096
L = 20
D = 64
VOCAB = 100000

FG = 2
NG = N_FIELDS // FG
SLAB_ROWS = FG * VOCAB

NC = 2
NS = 16
NW = NC * NS
BPW = BATCH // NW
CB = 32
CHUNKS_PER_FIELD = BPW // CB
TOT_CHUNKS = FG * CHUNKS_PER_FIELD
IPC = CB * L
IDX_ROWS = IPC // 128


def _sc_body(idx_hbm, table_hbm, offs_hbm, out_hbm,
             idx_v0, idx_v1, rows_v0, rows_v1, out_v, offs_v, sem0, sem1):
    c = lax.axis_index("c")
    s = lax.axis_index("s")
    wid = s * NC + c

    idx_vs = (idx_v0, idx_v1)
    rows_vs = (rows_v0, rows_v1)
    sems = (sem0, sem1)

    pltpu.sync_copy(offs_hbm, offs_v)

    def base_bag_of(t):
        f = t // CHUNKS_PER_FIELD
        cc = lax.rem(t, CHUNKS_PER_FIELD)
        return f, f * BATCH + wid * BPW + cc * CB

    def fire(t, ib):
        f, base_bag = base_bag_of(t)
        pltpu.sync_copy(idx_hbm.at[pl.ds(base_bag * L, IPC)], idx_vs[ib])
        off = offs_v[pl.ds(f * 16, 16)]
        for i in range(IPC // 16):
            sl = pl.ds(i * 16, 16)
            idx_vs[ib][sl] = idx_vs[ib][sl] + off
        for i in range(IDX_ROWS):
            pltpu.async_copy(table_hbm.at[idx_vs[ib].at[pl.ds(i * 128, 128)]],
                             rows_vs[ib].at[pl.ds(i * 128, 128)], sems[ib])

    def drain(ib):
        pltpu.make_async_copy(table_hbm.at[pl.ds(0, IPC)],
                              rows_vs[ib], sems[ib]).wait()

    def reduce_store(t, ib):
        _, base_bag = base_bag_of(t)
        rows_v = rows_vs[ib]

        @pl.loop(0, CB)
        def bag_loop(j):
            row0 = j * L
            for k in range(4):
                sl = pl.ds(k * 16, 16)
                a0 = rows_v[row0 + 0, sl] + rows_v[row0 + 1, sl]
                a1 = rows_v[row0 + 2, sl] + rows_v[row0 + 3, sl]
                a2 = rows_v[row0 + 4, sl] + rows_v[row0 + 5, sl]
                a3 = rows_v[row0 + 6, sl] + rows_v[row0 + 7, sl]
                for l in range(8, L, 4):
                    a0 = a0 + rows_v[row0 + l + 0, sl]
                    a1 = a1 + rows_v[row0 + l + 1, sl]
                    a2 = a2 + rows_v[row0 + l + 2, sl]
                    a3 = a3 + rows_v[row0 + l + 3, sl]
                out_v[j, sl] = (a0 + a1) + (a2 + a3)

        pltpu.sync_copy(out_v, out_hbm.at[pl.ds(base_bag, CB)])

    fire(0, 0)

    @pl.loop(0, TOT_CHUNKS, step=2)
    def chunk_loop(t):
        for b in range(2):
            tb = t + b
            nxt = tb + 1

            @pl.when(nxt < TOT_CHUNKS)
            def _():
                fire(nxt, 1 - b)

            drain(b)
            reduce_store(tb, b)


@jax.jit
def _fused_bag_sum(idx1d, table, offsets):
    mesh = plsc.VectorSubcoreMesh(core_axis_name="c", subcore_axis_name="s")
    fn = pl.kernel(
        _sc_body,
        out_type=jax.ShapeDtypeStruct((FG * BATCH, D), jnp.float32),
        mesh=mesh,
        compiler_params=pltpu.CompilerParams(use_tc_tiling_on_sc=False),
        scratch_types=[
            pltpu.VMEM((IPC,), jnp.int32),
            pltpu.VMEM((IPC,), jnp.int32),
            pltpu.VMEM((IPC, D), jnp.float32),
            pltpu.VMEM((IPC, D), jnp.float32),
            pltpu.VMEM((CB, D), jnp.float32),
            pltpu.VMEM((32,), jnp.int32),
            pltpu.SemaphoreType.DMA,
            pltpu.SemaphoreType.DMA,
        ],
    )
    tbl_rm = lax.optimization_barrier(
        jnp.swapaxes(lax.optimization_barrier(jnp.swapaxes(table, 0, 1)), 0, 1))
    outs = []
    for g in range(NG):
        idx_g = lax.slice(idx1d, (g * FG * BATCH * L,),
                          ((g + 1) * FG * BATCH * L,))
        tbl_g = lax.slice(tbl_rm, (g * SLAB_ROWS, 0), ((g + 1) * SLAB_ROWS, D))
        offs_g = jnp.repeat(
            lax.slice(offsets, (g * FG,), ((g + 1) * FG,)) - g * SLAB_ROWS,
            16,
        )
        outs.append(fn(idx_g, tbl_g, offs_g))
    return jnp.concatenate(outs, axis=0)


def kernel(indices, table, offsets):
    idx1d = indices.reshape(-1)
    out = _fused_bag_sum(idx1d, table, offsets)
    return out.reshape(BATCH, N_FIELDS, D)


# speedup vs baseline: 1.4102x; 1.4080x over previous
import functools

import jax
import jax.numpy as jnp
from jax import lax
from jax.experimental import pallas as pl
from jax.experimental.pallas import tpu as pltpu
from jax.experimental.pallas import tpu_sc as plsc

N_FIELDS = 26
BATCH = 4096
L = 20
D = 64

NC = 2
NS = 16
NW = NC * NS
BPW = BATCH // NW
CB = 32
CHUNKS_PER_FIELD = BPW // CB
TOT_CHUNKS = N_FIELDS * CHUNKS_PER_FIELD
IPC = CB * L
IDX_ROWS = IPC // 128


def _sc_body(idx_hbm, table_hbm, offs_hbm, out_hbm,
             idx_v0, idx_v1, rows_v0, rows_v1, out_v, offs_v,
             sem0, sem1, isem0, isem1):
    c = lax.axis_index("c")
    s = lax.axis_index("s")
    wid = s * NC + c

    idx_vs = (idx_v0, idx_v1)
    rows_vs = (rows_v0, rows_v1)
    sems = (sem0, sem1)
    isems = (isem0, isem1)

    pltpu.sync_copy(offs_hbm, offs_v)

    def base_bag_of(t):
        f = t // CHUNKS_PER_FIELD
        cc = lax.rem(t, CHUNKS_PER_FIELD)
        return f, f * BATCH + wid * BPW + cc * CB

    def fire_idx(t, ib):
        _, base_bag = base_bag_of(t)
        pltpu.async_copy(idx_hbm.at[pl.ds(base_bag * L, IPC)],
                         idx_vs[ib], isems[ib])

    def fire_gather(t, ib):
        f, _ = base_bag_of(t)
        pltpu.make_async_copy(idx_hbm.at[pl.ds(0, IPC)],
                              idx_vs[ib], isems[ib]).wait()
        off = offs_v[pl.ds(f * 16, 16)]
        for i in range(IPC // 16):
            sl = pl.ds(i * 16, 16)
            idx_vs[ib][sl] = idx_vs[ib][sl] + off
        for i in range(IDX_ROWS):
            pltpu.async_copy(table_hbm.at[idx_vs[ib].at[pl.ds(i * 128, 128)]],
                             rows_vs[ib].at[pl.ds(i * 128, 128)], sems[ib])

    def drain(ib):
        pltpu.make_async_copy(table_hbm.at[pl.ds(0, IPC)],
                              rows_vs[ib], sems[ib]).wait()

    def reduce_store(t, ib):
        _, base_bag = base_bag_of(t)
        rows_v = rows_vs[ib]

        @pl.loop(0, CB)
        def bag_loop(j):
            row0 = j * L
            for k in range(4):
                sl = pl.ds(k * 16, 16)
                a0 = rows_v[row0 + 0, sl] + rows_v[row0 + 1, sl]
                a1 = rows_v[row0 + 2, sl] + rows_v[row0 + 3, sl]
                a2 = rows_v[row0 + 4, sl] + rows_v[row0 + 5, sl]
                a3 = rows_v[row0 + 6, sl] + rows_v[row0 + 7, sl]
                for l in range(8, L, 4):
                    a0 = a0 + rows_v[row0 + l + 0, sl]
                    a1 = a1 + rows_v[row0 + l + 1, sl]
                    a2 = a2 + rows_v[row0 + l + 2, sl]
                    a3 = a3 + rows_v[row0 + l + 3, sl]
                out_v[j, sl] = (a0 + a1) + (a2 + a3)

        pltpu.sync_copy(out_v, out_hbm.at[pl.ds(base_bag, CB)])

    fire_idx(0, 0)
    fire_gather(0, 0)
    fire_idx(1, 1)

    @pl.loop(0, TOT_CHUNKS, step=2)
    def chunk_loop(t):
        for b in range(2):
            tb = t + b
            nxt = tb + 1

            @pl.when(nxt < TOT_CHUNKS)
            def _():
                fire_gather(nxt, 1 - b)

            drain(b)

            @pl.when(tb + 2 < TOT_CHUNKS)
            def _():
                fire_idx(tb + 2, b)

            reduce_store(tb, b)


@jax.jit
def _fused_bag_sum(idx1d, table, offs_rep):
    mesh = plsc.VectorSubcoreMesh(core_axis_name="c", subcore_axis_name="s")
    fn = pl.kernel(
        _sc_body,
        out_type=jax.ShapeDtypeStruct((N_FIELDS * BATCH, D), jnp.float32),
        mesh=mesh,
        compiler_params=pltpu.CompilerParams(use_tc_tiling_on_sc=False),
        scratch_types=[
            pltpu.VMEM((IPC,), jnp.int32),
            pltpu.VMEM((IPC,), jnp.int32),
            pltpu.VMEM((IPC, D), jnp.float32),
            pltpu.VMEM((IPC, D), jnp.float32),
            pltpu.VMEM((CB, D), jnp.float32),
            pltpu.VMEM((512,), jnp.int32),
            pltpu.SemaphoreType.DMA,
            pltpu.SemaphoreType.DMA,
            pltpu.SemaphoreType.DMA,
            pltpu.SemaphoreType.DMA,
        ],
    )
    return fn(idx1d, table, offs_rep)


def kernel(indices, table, offsets):
    idx1d = indices.reshape(-1)
    offs_rep = jnp.pad(jnp.repeat(offsets, 16), (0, 16 * (32 - N_FIELDS)))
    out = _fused_bag_sum(idx1d, table, offs_rep)
    return out.reshape(BATCH, N_FIELDS, D)


# speedup vs baseline: 1.4108x; 1.0004x over previous
import jax
import jax.numpy as jnp
from jax import lax
from jax.experimental import pallas as pl
from jax.experimental.pallas import tpu as pltpu
from jax.experimental.pallas import tpu_sc as plsc

N_FIELDS = 26
BATCH = 4096
L = 20
D = 64

NC = 2
NS = 16
NW = NC * NS
BPW = BATCH // NW
CB = 32
CHUNKS_PER_FIELD = BPW // CB
TOT_CHUNKS = N_FIELDS * CHUNKS_PER_FIELD
IPC = CB * L
IDX_ROWS = IPC // 128


def _sc_body(idx_hbm, table_hbm, offs_hbm, out_hbm,
             idx_v0, idx_v1, rows_v0, rows_v1, out_v, offs_v,
             sem0, sem1, isem0, isem1):
    c = lax.axis_index("c")
    s = lax.axis_index("s")
    wid = s * NC + c

    idx_vs = (idx_v0, idx_v1)
    rows_vs = (rows_v0, rows_v1)
    sems = (sem0, sem1)
    isems = (isem0, isem1)

    pltpu.sync_copy(offs_hbm, offs_v)

    def base_bag_of(t):
        f = t // CHUNKS_PER_FIELD
        cc = lax.rem(t, CHUNKS_PER_FIELD)
        return f, f * BATCH + wid * BPW + cc * CB

    def fire_idx(t, ib):
        _, base_bag = base_bag_of(t)
        pltpu.async_copy(idx_hbm.at[pl.ds(base_bag * L, IPC)],
                         idx_vs[ib], isems[ib])

    def fire_gather(t, ib):
        f, _ = base_bag_of(t)
        pltpu.make_async_copy(idx_hbm.at[pl.ds(0, IPC)],
                              idx_vs[ib], isems[ib]).wait()
        off = offs_v[pl.ds(f * 16, 16)]
        for i in range(IPC // 16):
            sl = pl.ds(i * 16, 16)
            idx_vs[ib][sl] = idx_vs[ib][sl] + off
        for i in range(IDX_ROWS):
            pltpu.async_copy(table_hbm.at[idx_vs[ib].at[pl.ds(i * 128, 128)]],
                             rows_vs[ib].at[pl.ds(i * 128, 128)], sems[ib])

    def drain(ib):
        pltpu.make_async_copy(table_hbm.at[pl.ds(0, IPC)],
                              rows_vs[ib], sems[ib]).wait()

    def reduce_store(t, ib):
        _, base_bag = base_bag_of(t)
        rows_v = rows_vs[ib]

        @pl.loop(0, CB)
        def bag_loop(j):
            row0 = j * L
            for k in range(4):
                sl = pl.ds(k * 16, 16)
                a0 = rows_v[row0 + 0, sl] + rows_v[row0 + 1, sl]
                a1 = rows_v[row0 + 2, sl] + rows_v[row0 + 3, sl]
                a2 = rows_v[row0 + 4, sl] + rows_v[row0 + 5, sl]
                a3 = rows_v[row0 + 6, sl] + rows_v[row0 + 7, sl]
                for l in range(8, L, 4):
                    a0 = a0 + rows_v[row0 + l + 0, sl]
                    a1 = a1 + rows_v[row0 + l + 1, sl]
                    a2 = a2 + rows_v[row0 + l + 2, sl]
                    a3 = a3 + rows_v[row0 + l + 3, sl]
                out_v[j, sl] = (a0 + a1) + (a2 + a3)

        pltpu.sync_copy(out_v, out_hbm.at[pl.ds(base_bag, CB)])

    fire_idx(0, 0)
    fire_gather(0, 0)
    fire_idx(1, 1)

    @pl.loop(0, TOT_CHUNKS, step=2)
    def chunk_loop(t):
        for b in range(2):
            tb = t + b
            nxt = tb + 1

            @pl.when(nxt < TOT_CHUNKS)
            def _():
                fire_gather(nxt, 1 - b)

            drain(b)

            @pl.when(tb + 2 < TOT_CHUNKS)
            def _():
                fire_idx(tb + 2, b)

            reduce_store(tb, b)


@jax.jit
def _fused_bag_sum(idx1d, table, offs_rep):
    mesh = plsc.VectorSubcoreMesh(core_axis_name="c", subcore_axis_name="s")
    fn = pl.kernel(
        _sc_body,
        out_type=jax.ShapeDtypeStruct((N_FIELDS * BATCH, D), jnp.float32),
        mesh=mesh,
        compiler_params=pltpu.CompilerParams(use_tc_tiling_on_sc=False),
        scratch_types=[
            pltpu.VMEM((IPC,), jnp.int32),
            pltpu.VMEM((IPC,), jnp.int32),
            pltpu.VMEM((IPC, D), jnp.float32),
            pltpu.VMEM((IPC, D), jnp.float32),
            pltpu.VMEM((CB, D), jnp.float32),
            pltpu.VMEM((512,), jnp.int32),
            pltpu.SemaphoreType.DMA,
            pltpu.SemaphoreType.DMA,
            pltpu.SemaphoreType.DMA,
            pltpu.SemaphoreType.DMA,
        ],
    )
    return fn(idx1d, table, offs_rep)


def kernel(indices, table, offsets):
    idx1d = indices.reshape(-1)
    offs_rep = jnp.pad(jnp.repeat(offsets, 16), (0, 16 * (32 - N_FIELDS)))
    out = _fused_bag_sum(idx1d, table, offs_rep)
    return out.reshape(BATCH, N_FIELDS, D)
